# baseline (device time: 141096 ns/iter reference)
import jax
import jax.numpy as jnp
from jax import lax
from jax.experimental import pallas as pl
from jax.experimental.pallas import tpu as pltpu

N_DEV = 8
SQ = 1024
HQ = 8
DH = 128
D_MODEL = 1024
CH = SQ // N_DEV
SCALE = 0.08838834764831843


def _body(x_ref, wq_ref, k_ref, v_ref, wo_ref, out_ref,
          acc_ref, ctx_ref, recv_ref,
          rs_send_sems, rs_recv_sems, ag_send_sems, ag_recv_sems):
    my = lax.axis_index("i")
    left = lax.rem(my + N_DEV - 1, N_DEV)
    right = lax.rem(my + 1, N_DEV)

    barrier = pltpu.get_barrier_semaphore()
    for nbr in (left, right):
        pl.semaphore_signal(barrier, inc=1, device_id=(nbr,),
                            device_id_type=pl.DeviceIdType.MESH)
    pl.semaphore_wait(barrier, 2)

    q = jnp.dot(x_ref[...], wq_ref[...],
                preferred_element_type=jnp.float32) * SCALE
    row = lax.broadcasted_iota(jnp.int32, (SQ, SQ), 0)
    col = lax.broadcasted_iota(jnp.int32, (SQ, SQ), 1)
    mask = ((row // 64) % 4) == ((col // 64) % 4)
    for h in range(HQ):
        qh = q[:, h * DH:(h + 1) * DH].astype(jnp.bfloat16)
        s = lax.dot_general(qh, k_ref[h], (((1,), (1,)), ((), ())),
                            preferred_element_type=jnp.float32)
        s = jnp.where(mask, s, -1e9)
        w = jnp.exp(s - jnp.max(s, axis=1, keepdims=True))
        w = (w / jnp.sum(w, axis=1, keepdims=True)).astype(jnp.bfloat16)
        ctx_ref[:, h * DH:(h + 1) * DH] = jnp.dot(
            w, v_ref[h], preferred_element_type=jnp.float32
        ).astype(jnp.bfloat16)
    acc_ref[...] = jnp.dot(ctx_ref[...], wo_ref[...],
                           preferred_element_type=jnp.float32)

    for step in range(N_DEV - 1):
        send_c = lax.rem(my - step + N_DEV, N_DEV)
        rdma = pltpu.make_async_remote_copy(
            src_ref=acc_ref.at[pl.ds(send_c * CH, CH), :],
            dst_ref=recv_ref.at[step],
            send_sem=rs_send_sems.at[step],
            recv_sem=rs_recv_sems.at[step],
            device_id=(right,),
            device_id_type=pl.DeviceIdType.MESH,
        )
        rdma.start()
        rdma.wait()
        recv_c = lax.rem(my - step - 1 + N_DEV, N_DEV)
        sl = pl.ds(recv_c * CH, CH)
        acc_ref[sl, :] = acc_ref[sl, :] + recv_ref[step]

    own = lax.rem(my + 1, N_DEV)
    sl = pl.ds(own * CH, CH)
    out_ref[sl, :] = acc_ref[sl, :]

    for t in range(N_DEV - 1):
        send_c = lax.rem(my + 1 - t + N_DEV, N_DEV)
        sl = pl.ds(send_c * CH, CH)
        rdma = pltpu.make_async_remote_copy(
            src_ref=out_ref.at[sl, :],
            dst_ref=out_ref.at[sl, :],
            send_sem=ag_send_sems.at[t],
            recv_sem=ag_recv_sems.at[t],
            device_id=(right,),
            device_id_type=pl.DeviceIdType.MESH,
        )
        rdma.start()
        rdma.wait()


def kernel(x, Wq, K_ext, V_ext, Wo):
    my = lax.axis_index("i")
    xb = x[0].astype(jnp.bfloat16)
    Wqb = Wq.astype(jnp.bfloat16)
    h0 = my * HQ
    Kh = lax.dynamic_slice_in_dim(K_ext[0], h0, HQ, axis=1)
    Vh = lax.dynamic_slice_in_dim(V_ext[0], h0, HQ, axis=1)
    Kb = jnp.transpose(Kh, (1, 0, 2)).astype(jnp.bfloat16)
    Vb = jnp.transpose(Vh, (1, 0, 2)).astype(jnp.bfloat16)
    Wob = Wo.astype(jnp.bfloat16)

    out = pl.pallas_call(
        _body,
        out_shape=jax.ShapeDtypeStruct((SQ, D_MODEL), jnp.float32),
        in_specs=[pl.BlockSpec(memory_space=pltpu.VMEM)] * 5,
        out_specs=pl.BlockSpec(memory_space=pltpu.VMEM),
        scratch_shapes=[
            pltpu.VMEM((SQ, D_MODEL), jnp.float32),
            pltpu.VMEM((SQ, HQ * DH), jnp.bfloat16),
            pltpu.VMEM((N_DEV - 1, CH, D_MODEL), jnp.float32),
            pltpu.SemaphoreType.DMA((N_DEV - 1,)),
            pltpu.SemaphoreType.DMA((N_DEV - 1,)),
            pltpu.SemaphoreType.DMA((N_DEV - 1,)),
            pltpu.SemaphoreType.DMA((N_DEV - 1,)),
        ],
        compiler_params=pltpu.CompilerParams(collective_id=0),
    )(xb, Wqb, Kb, Vb, Wob)
    return out[None]


# device time: 68962 ns/iter; 2.0460x vs baseline; 2.0460x over previous
import jax
import jax.numpy as jnp
from jax import lax
from jax.experimental import pallas as pl
from jax.experimental.pallas import tpu as pltpu

N_DEV = 8
SQ = 1024
HQ = 8
DH = 128
D_MODEL = 1024
HALF = SQ // 2
SCALE = 0.08838834764831843

DIMS = ((0, 1, 2), (1, 2, 0))
SEG = tuple(HALF >> (r + 1) for r in range(3))


def _phys(v):
    low = v & 3
    return (v & 4) | (low ^ (low >> 1))


def _body(x_ref, wq_ref, k_ref, v_ref, wo_ref, out_ref,
          acc_ref, ctx_ref,
          ra0_ref, ra1_ref, ra2_ref, rb0_ref, rb1_ref, rb2_ref,
          rs_send, rs_recv, ag_send, ag_recv):
    my = lax.axis_index("i")
    my_v = _phys(my)
    partners = [_phys(my_v ^ (1 << b)) for b in range(3)]

    barrier = pltpu.get_barrier_semaphore()
    for nbr in partners:
        pl.semaphore_signal(barrier, inc=1, device_id=(nbr,),
                            device_id_type=pl.DeviceIdType.MESH)
    pl.semaphore_wait(barrier, 3)

    recv_bufs = ((ra0_ref, ra1_ref, ra2_ref), (rb0_ref, rb1_ref, rb2_ref))

    lo = [jnp.int32(0), jnp.int32(HALF)]

    def rs_rdma(h, r):
        seg = SEG[r]
        bit = DIMS[h][r]
        keep = (my_v >> bit) & 1
        send_off = lo[h] + (1 - keep) * seg
        lo[h] = lo[h] + keep * seg
        return pltpu.make_async_remote_copy(
            src_ref=acc_ref.at[pl.ds(send_off, seg), :],
            dst_ref=recv_bufs[h][r],
            send_sem=rs_send.at[h * 3 + r],
            recv_sem=rs_recv.at[h * 3 + r],
            device_id=(partners[bit],),
            device_id_type=pl.DeviceIdType.MESH,
        )

    def rs_add(h, r):
        seg = SEG[r]
        sl = pl.ds(lo[h], seg)
        acc_ref[sl, :] = acc_ref[sl, :] + recv_bufs[h][r][...]

    def ag_rdma(h, r):
        seg = SEG[r] if r == 2 else 2 * SEG[r + 1]
        sl = pl.ds(lo[h], seg)
        bit = DIMS[h][r]
        rdma = pltpu.make_async_remote_copy(
            src_ref=out_ref.at[sl, :],
            dst_ref=out_ref.at[sl, :],
            send_sem=ag_send.at[h * 3 + r],
            recv_sem=ag_recv.at[h * 3 + r],
            device_id=(partners[bit],),
            device_id_type=pl.DeviceIdType.MESH,
        )
        lo[h] = lo[h] - ((my_v >> bit) & 1) * (SEG[2] << (2 - r))
        return rdma

    q = jnp.dot(x_ref[...], wq_ref[...],
                preferred_element_type=jnp.float32) * SCALE
    col = lax.broadcasted_iota(jnp.int32, (HALF, SQ), 1)
    row = lax.broadcasted_iota(jnp.int32, (HALF, SQ), 0)

    rs = [None, None, None, None, None, None]

    for h_half in range(2):
        base = h_half * HALF
        mask = (((row + base) // 64) % 4) == ((col // 64) % 4)
        for h in range(HQ):
            qh = q[base:base + HALF, h * DH:(h + 1) * DH].astype(jnp.bfloat16)
            s = lax.dot_general(qh, k_ref[h], (((1,), (1,)), ((), ())),
                                preferred_element_type=jnp.float32)
            s = jnp.where(mask, s, -1e9)
            w = jnp.exp(s - jnp.max(s, axis=1, keepdims=True))
            w = (w / jnp.sum(w, axis=1, keepdims=True)).astype(jnp.bfloat16)
            ctx_ref[base:base + HALF, h * DH:(h + 1) * DH] = jnp.dot(
                w, v_ref[h], preferred_element_type=jnp.float32
            ).astype(jnp.bfloat16)
        acc_ref[base:base + HALF, :] = jnp.dot(
            ctx_ref[base:base + HALF, :], wo_ref[...],
            preferred_element_type=jnp.float32).astype(jnp.bfloat16)
        rs[h_half * 3] = rs_rdma(h_half, 0)
        rs[h_half * 3].start()

    for r in range(3):
        for h_half in range(2):
            rs[h_half * 3 + r].wait()
            rs_add(h_half, r)
            if r < 2:
                rs[h_half * 3 + r + 1] = rs_rdma(h_half, r + 1)
                rs[h_half * 3 + r + 1].start()

    ag = [None] * 6
    for h_half in range(2):
        sl = pl.ds(lo[h_half], SEG[2])
        out_ref[sl, :] = acc_ref[sl, :]
        ag[h_half * 3 + 2] = ag_rdma(h_half, 2)
        ag[h_half * 3 + 2].start()
    for r in (2, 1, 0):
        for h_half in range(2):
            ag[h_half * 3 + r].wait()
            if r > 0:
                ag[h_half * 3 + r - 1] = ag_rdma(h_half, r - 1)
                ag[h_half * 3 + r - 1].start()


def kernel(x, Wq, K_ext, V_ext, Wo):
    my = lax.axis_index("i")
    xb = x[0].astype(jnp.bfloat16)
    Wqb = Wq.astype(jnp.bfloat16)
    h0 = my * HQ
    Kh = lax.dynamic_slice_in_dim(K_ext[0], h0, HQ, axis=1)
    Vh = lax.dynamic_slice_in_dim(V_ext[0], h0, HQ, axis=1)
    Kb = jnp.transpose(Kh, (1, 0, 2)).astype(jnp.bfloat16)
    Vb = jnp.transpose(Vh, (1, 0, 2)).astype(jnp.bfloat16)
    Wob = Wo.astype(jnp.bfloat16)

    out = pl.pallas_call(
        _body,
        out_shape=jax.ShapeDtypeStruct((SQ, D_MODEL), jnp.bfloat16),
        in_specs=[pl.BlockSpec(memory_space=pltpu.VMEM)] * 5,
        out_specs=pl.BlockSpec(memory_space=pltpu.VMEM),
        scratch_shapes=[
            pltpu.VMEM((SQ, D_MODEL), jnp.bfloat16),
            pltpu.VMEM((SQ, HQ * DH), jnp.bfloat16),
            pltpu.VMEM((SEG[0], D_MODEL), jnp.bfloat16),
            pltpu.VMEM((SEG[1], D_MODEL), jnp.bfloat16),
            pltpu.VMEM((SEG[2], D_MODEL), jnp.bfloat16),
            pltpu.VMEM((SEG[0], D_MODEL), jnp.bfloat16),
            pltpu.VMEM((SEG[1], D_MODEL), jnp.bfloat16),
            pltpu.VMEM((SEG[2], D_MODEL), jnp.bfloat16),
            pltpu.SemaphoreType.DMA((6,)),
            pltpu.SemaphoreType.DMA((6,)),
            pltpu.SemaphoreType.DMA((6,)),
            pltpu.SemaphoreType.DMA((6,)),
        ],
        compiler_params=pltpu.CompilerParams(collective_id=0),
    )(xb, Wqb, Kb, Vb, Wob)
    return out[None].astype(jnp.float32)


# device time: 58151 ns/iter; 2.4264x vs baseline; 1.1859x over previous
import jax
import jax.numpy as jnp
from jax import lax
from jax.experimental import pallas as pl
from jax.experimental.pallas import tpu as pltpu

N_DEV = 8
SQ = 1024
HQ = 8
DH = 128
D_MODEL = 1024
HALF = SQ // 2
SCALE = 0.08838834764831843

DIMS = ((0, 1, 2), (1, 2, 0))
SEG = tuple(HALF >> (r + 1) for r in range(3))


def _phys(v):
    low = v & 3
    return (v & 4) | (low ^ (low >> 1))


def _body(x_ref, wq_ref, k_ref, v_ref, wo_ref, out_ref,
          acc_ref, ctx_ref,
          ra0_ref, ra1_ref, ra2_ref, rb0_ref, rb1_ref, rb2_ref,
          rs_send, rs_recv, ag_send, ag_recv):
    my = lax.axis_index("i")
    my_v = _phys(my)
    partners = [_phys(my_v ^ (1 << b)) for b in range(3)]

    barrier = pltpu.get_barrier_semaphore()
    for nbr in partners:
        pl.semaphore_signal(barrier, inc=1, device_id=(nbr,),
                            device_id_type=pl.DeviceIdType.MESH)
    pl.semaphore_wait(barrier, 3)

    recv_bufs = ((ra0_ref, ra1_ref, ra2_ref), (rb0_ref, rb1_ref, rb2_ref))

    lo = [jnp.int32(0), jnp.int32(HALF)]

    def rs_rdma(h, r):
        seg = SEG[r]
        bit = DIMS[h][r]
        keep = (my_v >> bit) & 1
        send_off = lo[h] + (1 - keep) * seg
        lo[h] = lo[h] + keep * seg
        return pltpu.make_async_remote_copy(
            src_ref=acc_ref.at[pl.ds(send_off, seg), :],
            dst_ref=recv_bufs[h][r],
            send_sem=rs_send.at[h * 3 + r],
            recv_sem=rs_recv.at[h * 3 + r],
            device_id=(partners[bit],),
            device_id_type=pl.DeviceIdType.MESH,
        )

    def rs_add(h, r):
        seg = SEG[r]
        sl = pl.ds(lo[h], seg)
        acc_ref[sl, :] = acc_ref[sl, :] + recv_bufs[h][r][...]

    def ag_rdma(h, r):
        seg = SEG[r] if r == 2 else 2 * SEG[r + 1]
        sl = pl.ds(lo[h], seg)
        bit = DIMS[h][r]
        rdma = pltpu.make_async_remote_copy(
            src_ref=out_ref.at[sl, :],
            dst_ref=out_ref.at[sl, :],
            send_sem=ag_send.at[h * 3 + r],
            recv_sem=ag_recv.at[h * 3 + r],
            device_id=(partners[bit],),
            device_id_type=pl.DeviceIdType.MESH,
        )
        lo[h] = lo[h] - ((my_v >> bit) & 1) * (SEG[2] << (2 - r))
        return rdma

    q = jnp.dot(x_ref[...], wq_ref[...],
                preferred_element_type=jnp.float32) * SCALE

    def class_rows(mat, c, base):
        return jnp.concatenate(
            [mat[base + 64 * c:base + 64 * c + 64, :],
             mat[base + 64 * c + 256:base + 64 * c + 320, :]], axis=0)

    kc = [jnp.concatenate([k_ref[64 * c + 256 * j:64 * c + 256 * j + 64, :]
                           for j in range(4)], axis=0) for c in range(4)]
    vc = [jnp.concatenate([v_ref[64 * c + 256 * j:64 * c + 256 * j + 64, :]
                           for j in range(4)], axis=0) for c in range(4)]

    rs = [None, None, None, None, None, None]

    for h_half in range(2):
        base = h_half * HALF
        for c in range(4):
            qc = class_rows(q, c, base)
            for h in range(HQ):
                qh = qc[:, h * DH:(h + 1) * DH].astype(jnp.bfloat16)
                kh = kc[c][:, h * DH:(h + 1) * DH]
                s = lax.dot_general(qh, kh, (((1,), (1,)), ((), ())),
                                    preferred_element_type=jnp.float32)
                w = jnp.exp(s - jnp.max(s, axis=1, keepdims=True))
                w = (w / jnp.sum(w, axis=1, keepdims=True)).astype(jnp.bfloat16)
                ctx = jnp.dot(w, vc[c][:, h * DH:(h + 1) * DH],
                              preferred_element_type=jnp.float32
                              ).astype(jnp.bfloat16)
                hc = slice(h * DH, (h + 1) * DH)
                ctx_ref[base + 64 * c:base + 64 * c + 64, hc] = ctx[:64, :]
                ctx_ref[base + 64 * c + 256:base + 64 * c + 320, hc] = ctx[64:, :]
        acc_ref[base:base + HALF, :] = jnp.dot(
            ctx_ref[base:base + HALF, :], wo_ref[...],
            preferred_element_type=jnp.float32).astype(jnp.bfloat16)
        rs[h_half * 3] = rs_rdma(h_half, 0)
        rs[h_half * 3].start()

    for r in range(3):
        for h_half in range(2):
            rs[h_half * 3 + r].wait()
            rs_add(h_half, r)
            if r < 2:
                rs[h_half * 3 + r + 1] = rs_rdma(h_half, r + 1)
                rs[h_half * 3 + r + 1].start()

    ag = [None] * 6
    for h_half in range(2):
        sl = pl.ds(lo[h_half], SEG[2])
        out_ref[sl, :] = acc_ref[sl, :]
        ag[h_half * 3 + 2] = ag_rdma(h_half, 2)
        ag[h_half * 3 + 2].start()
    for r in (2, 1, 0):
        for h_half in range(2):
            ag[h_half * 3 + r].wait()
            if r > 0:
                ag[h_half * 3 + r - 1] = ag_rdma(h_half, r - 1)
                ag[h_half * 3 + r - 1].start()


def kernel(x, Wq, K_ext, V_ext, Wo):
    my = lax.axis_index("i")
    xb = x[0].astype(jnp.bfloat16)
    Wqb = Wq.astype(jnp.bfloat16)
    h0 = my * HQ
    Kh = lax.dynamic_slice_in_dim(K_ext[0], h0, HQ, axis=1)
    Vh = lax.dynamic_slice_in_dim(V_ext[0], h0, HQ, axis=1)
    Kb = Kh.reshape(SQ, HQ * DH).astype(jnp.bfloat16)
    Vb = Vh.reshape(SQ, HQ * DH).astype(jnp.bfloat16)
    Wob = Wo.astype(jnp.bfloat16)

    out = pl.pallas_call(
        _body,
        out_shape=jax.ShapeDtypeStruct((SQ, D_MODEL), jnp.bfloat16),
        in_specs=[pl.BlockSpec(memory_space=pltpu.VMEM)] * 5,
        out_specs=pl.BlockSpec(memory_space=pltpu.VMEM),
        scratch_shapes=[
            pltpu.VMEM((SQ, D_MODEL), jnp.bfloat16),
            pltpu.VMEM((SQ, HQ * DH), jnp.bfloat16),
            pltpu.VMEM((SEG[0], D_MODEL), jnp.bfloat16),
            pltpu.VMEM((SEG[1], D_MODEL), jnp.bfloat16),
            pltpu.VMEM((SEG[2], D_MODEL), jnp.bfloat16),
            pltpu.VMEM((SEG[0], D_MODEL), jnp.bfloat16),
            pltpu.VMEM((SEG[1], D_MODEL), jnp.bfloat16),
            pltpu.VMEM((SEG[2], D_MODEL), jnp.bfloat16),
            pltpu.SemaphoreType.DMA((6,)),
            pltpu.SemaphoreType.DMA((6,)),
            pltpu.SemaphoreType.DMA((6,)),
            pltpu.SemaphoreType.DMA((6,)),
        ],
        compiler_params=pltpu.CompilerParams(collective_id=0),
    )(xb, Wqb, Kb, Vb, Wob)
    return out[None]


# device time: 52219 ns/iter; 2.7020x vs baseline; 1.1136x over previous
import jax
import jax.numpy as jnp
from jax import lax
from jax.experimental import pallas as pl
from jax.experimental.pallas import tpu as pltpu

N_DEV = 8
SQ = 1024
HQ = 8
DH = 128
D_MODEL = 1024
HALF = SQ // 2
SCALE = 0.08838834764831843

DIMS = ((0, 1, 2), (1, 2, 0))
SEG = tuple(HALF >> (r + 1) for r in range(3))


def _phys(v):
    low = v & 3
    return (v & 4) | (low ^ (low >> 1))


def _body(x_ref, wq_ref, k_hbm, v_hbm, wo_ref, out_ref,
          acc_ref, ctx_ref, k_ref, v_ref,
          ra0_ref, ra1_ref, ra2_ref, rb0_ref, rb1_ref, rb2_ref,
          k_sems, v_sems, rs_send, rs_recv, ag_send, ag_recv):
    my = lax.axis_index("i")
    my_v = _phys(my)
    partners = [_phys(my_v ^ (1 << b)) for b in range(3)]

    h0 = my * HQ
    kv_copies = []
    for j in range(HQ):
        for hbm, vmem, sems in ((k_hbm, k_ref, k_sems), (v_hbm, v_ref, v_sems)):
            cp = pltpu.make_async_copy(hbm.at[0, :, h0 + j, :],
                                       vmem.at[j], sems.at[j])
            cp.start()
            kv_copies.append(cp)

    barrier = pltpu.get_barrier_semaphore()
    for nbr in partners:
        pl.semaphore_signal(barrier, inc=1, device_id=(nbr,),
                            device_id_type=pl.DeviceIdType.MESH)
    pl.semaphore_wait(barrier, 3)

    recv_bufs = ((ra0_ref, ra1_ref, ra2_ref), (rb0_ref, rb1_ref, rb2_ref))

    lo = [jnp.int32(0), jnp.int32(HALF)]

    def rs_rdma(h, r):
        seg = SEG[r]
        bit = DIMS[h][r]
        keep = (my_v >> bit) & 1
        send_off = lo[h] + (1 - keep) * seg
        lo[h] = lo[h] + keep * seg
        return pltpu.make_async_remote_copy(
            src_ref=acc_ref.at[pl.ds(send_off, seg), :],
            dst_ref=recv_bufs[h][r],
            send_sem=rs_send.at[h * 3 + r],
            recv_sem=rs_recv.at[h * 3 + r],
            device_id=(partners[bit],),
            device_id_type=pl.DeviceIdType.MESH,
        )

    def rs_add(h, r):
        seg = SEG[r]
        sl = pl.ds(lo[h], seg)
        acc_ref[sl, :] = acc_ref[sl, :] + recv_bufs[h][r][...]

    def ag_rdma(h, r):
        seg = SEG[r] if r == 2 else 2 * SEG[r + 1]
        sl = pl.ds(lo[h], seg)
        bit = DIMS[h][r]
        rdma = pltpu.make_async_remote_copy(
            src_ref=out_ref.at[sl, :],
            dst_ref=out_ref.at[sl, :],
            send_sem=ag_send.at[h * 3 + r],
            recv_sem=ag_recv.at[h * 3 + r],
            device_id=(partners[bit],),
            device_id_type=pl.DeviceIdType.MESH,
        )
        lo[h] = lo[h] - ((my_v >> bit) & 1) * (SEG[2] << (2 - r))
        return rdma

    q = jnp.dot(x_ref[...].astype(jnp.bfloat16),
                wq_ref[...].astype(jnp.bfloat16),
                preferred_element_type=jnp.float32) * SCALE
    wo_b = wo_ref[...].astype(jnp.bfloat16)

    for cp in kv_copies:
        cp.wait()

    def class_rows(mat, c, base):
        return jnp.concatenate(
            [mat[base + 64 * c:base + 64 * c + 64, :],
             mat[base + 64 * c + 256:base + 64 * c + 320, :]], axis=0)

    def class_keys(ref, c, h):
        return jnp.concatenate(
            [ref[h, 64 * c + 256 * j:64 * c + 256 * j + 64, :]
             for j in range(4)], axis=0).astype(jnp.bfloat16)

    rs = [None, None, None, None, None, None]

    for h_half in range(2):
        base = h_half * HALF
        for c in range(4):
            qc = class_rows(q, c, base)
            for h in range(HQ):
                qh = qc[:, h * DH:(h + 1) * DH].astype(jnp.bfloat16)
                s = lax.dot_general(qh, class_keys(k_ref, c, h),
                                    (((1,), (1,)), ((), ())),
                                    preferred_element_type=jnp.float32)
                w = jnp.exp(s - jnp.max(s, axis=1, keepdims=True))
                w = (w / jnp.sum(w, axis=1, keepdims=True)).astype(jnp.bfloat16)
                ctx = jnp.dot(w, class_keys(v_ref, c, h),
                              preferred_element_type=jnp.float32
                              ).astype(jnp.bfloat16)
                hc = slice(h * DH, (h + 1) * DH)
                ctx_ref[base + 64 * c:base + 64 * c + 64, hc] = ctx[:64, :]
                ctx_ref[base + 64 * c + 256:base + 64 * c + 320, hc] = ctx[64:, :]
        acc_ref[base:base + HALF, :] = jnp.dot(
            ctx_ref[base:base + HALF, :], wo_b,
            preferred_element_type=jnp.float32).astype(jnp.bfloat16)
        rs[h_half * 3] = rs_rdma(h_half, 0)
        rs[h_half * 3].start()

    for r in range(3):
        for h_half in range(2):
            rs[h_half * 3 + r].wait()
            rs_add(h_half, r)
            if r < 2:
                rs[h_half * 3 + r + 1] = rs_rdma(h_half, r + 1)
                rs[h_half * 3 + r + 1].start()

    ag = [None] * 6
    for h_half in range(2):
        sl = pl.ds(lo[h_half], SEG[2])
        out_ref[sl, :] = acc_ref[sl, :]
        ag[h_half * 3 + 2] = ag_rdma(h_half, 2)
        ag[h_half * 3 + 2].start()
    for r in (2, 1, 0):
        for h_half in range(2):
            ag[h_half * 3 + r].wait()
            if r > 0:
                ag[h_half * 3 + r - 1] = ag_rdma(h_half, r - 1)
                ag[h_half * 3 + r - 1].start()


def kernel(x, Wq, K_ext, V_ext, Wo):
    out = pl.pallas_call(
        _body,
        out_shape=jax.ShapeDtypeStruct((SQ, D_MODEL), jnp.bfloat16),
        in_specs=[
            pl.BlockSpec(memory_space=pltpu.VMEM),
            pl.BlockSpec(memory_space=pltpu.VMEM),
            pl.BlockSpec(memory_space=pl.ANY),
            pl.BlockSpec(memory_space=pl.ANY),
            pl.BlockSpec(memory_space=pltpu.VMEM),
        ],
        out_specs=pl.BlockSpec(memory_space=pltpu.VMEM),
        scratch_shapes=[
            pltpu.VMEM((SQ, D_MODEL), jnp.bfloat16),
            pltpu.VMEM((SQ, HQ * DH), jnp.bfloat16),
            pltpu.VMEM((HQ, SQ, DH), jnp.float32),
            pltpu.VMEM((HQ, SQ, DH), jnp.float32),
            pltpu.VMEM((SEG[0], D_MODEL), jnp.bfloat16),
            pltpu.VMEM((SEG[1], D_MODEL), jnp.bfloat16),
            pltpu.VMEM((SEG[2], D_MODEL), jnp.bfloat16),
            pltpu.VMEM((SEG[0], D_MODEL), jnp.bfloat16),
            pltpu.VMEM((SEG[1], D_MODEL), jnp.bfloat16),
            pltpu.VMEM((SEG[2], D_MODEL), jnp.bfloat16),
            pltpu.SemaphoreType.DMA((HQ,)),
            pltpu.SemaphoreType.DMA((HQ,)),
            pltpu.SemaphoreType.DMA((6,)),
            pltpu.SemaphoreType.DMA((6,)),
            pltpu.SemaphoreType.DMA((6,)),
            pltpu.SemaphoreType.DMA((6,)),
        ],
        compiler_params=pltpu.CompilerParams(collective_id=0),
    )(x[0], Wq, K_ext, V_ext, Wo)
    return out[None]


# device time: 46922 ns/iter; 3.0070x vs baseline; 1.1129x over previous
import jax
import jax.numpy as jnp
from jax import lax
from jax.experimental import pallas as pl
from jax.experimental.pallas import tpu as pltpu

N_DEV = 8
SQ = 1024
HQ = 8
DH = 128
D_MODEL = 1024
HALF = SQ // 2
SCALE = 0.08838834764831843

BASE = (0, 384, 768)
PSZ = (384, 384, 256)
DIMS = ((0, 1, 2), (1, 2, 0), (2, 0, 1))
SEG = tuple(tuple(PSZ[p] >> (r + 1) for r in range(3)) for p in range(3))


def _phys(v):
    low = v & 3
    return (v & 4) | (low ^ (low >> 1))


def _body(x_ref, wq_ref, k_hbm, v_hbm, wo_ref, out_ref,
          acc_ref, ctx_ref, k_ref, v_ref,
          r00, r01, r02, r10, r11, r12, r20, r21, r22,
          k_sems, v_sems, rs_send, rs_recv, ag_send, ag_recv):
    my = lax.axis_index("i")
    my_v = _phys(my)
    partners = [_phys(my_v ^ (1 << b)) for b in range(3)]

    h0 = my * HQ
    kv_copies = []
    for j in range(HQ):
        for hbm, vmem, sems in ((k_hbm, k_ref, k_sems), (v_hbm, v_ref, v_sems)):
            cp = pltpu.make_async_copy(hbm.at[0, :, h0 + j, :],
                                       vmem.at[j], sems.at[j])
            cp.start()
            kv_copies.append(cp)

    barrier = pltpu.get_barrier_semaphore()
    for nbr in partners:
        pl.semaphore_signal(barrier, inc=1, device_id=(nbr,),
                            device_id_type=pl.DeviceIdType.MESH)
    pl.semaphore_wait(barrier, 3)

    recv_bufs = ((r00, r01, r02), (r10, r11, r12), (r20, r21, r22))
    lo = [jnp.int32(BASE[0]), jnp.int32(BASE[1]), jnp.int32(BASE[2])]

    def rs_rdma(p, r):
        seg = SEG[p][r]
        bit = DIMS[p][r]
        keep = (my_v >> bit) & 1
        send_off = lo[p] + (1 - keep) * seg
        lo[p] = lo[p] + keep * seg
        return pltpu.make_async_remote_copy(
            src_ref=acc_ref.at[pl.ds(send_off, seg), :],
            dst_ref=recv_bufs[p][r],
            send_sem=rs_send.at[p * 3 + r],
            recv_sem=rs_recv.at[p * 3 + r],
            device_id=(partners[bit],),
            device_id_type=pl.DeviceIdType.MESH,
        )

    def rs_add(p, r):
        sl = pl.ds(lo[p], SEG[p][r])
        acc_ref[sl, :] = acc_ref[sl, :] + recv_bufs[p][r][...]

    def ag_rdma(p, r):
        seg = SEG[p][2] << (2 - r)
        sl = pl.ds(lo[p], seg)
        bit = DIMS[p][r]
        rdma = pltpu.make_async_remote_copy(
            src_ref=out_ref.at[sl, :],
            dst_ref=out_ref.at[sl, :],
            send_sem=ag_send.at[p * 3 + r],
            recv_sem=ag_recv.at[p * 3 + r],
            device_id=(partners[bit],),
            device_id_type=pl.DeviceIdType.MESH,
        )
        lo[p] = lo[p] - ((my_v >> bit) & 1) * seg
        return rdma

    q = jnp.dot(x_ref[...].astype(jnp.bfloat16),
                wq_ref[...].astype(jnp.bfloat16),
                preferred_element_type=jnp.float32) * SCALE
    wo_b = wo_ref[...].astype(jnp.bfloat16)

    for cp in kv_copies:
        cp.wait()

    kc = [jnp.concatenate([k_ref[:, 64 * c + 256 * j:64 * c + 256 * j + 64, :]
                           for j in range(4)], axis=1).astype(jnp.bfloat16)
          for c in range(4)]
    vc = [jnp.concatenate([v_ref[:, 64 * c + 256 * j:64 * c + 256 * j + 64, :]
                           for j in range(4)], axis=1).astype(jnp.bfloat16)
          for c in range(4)]

    def class_rows(mat, c, base):
        return jnp.concatenate(
            [mat[base + 64 * c:base + 64 * c + 64, :],
             mat[base + 64 * c + 256:base + 64 * c + 320, :]], axis=0)

    rs = [None] * 9

    for h_half in range(2):
        base = h_half * HALF
        for c in range(4):
            qc = class_rows(q, c, base)
            for h in range(HQ):
                qh = qc[:, h * DH:(h + 1) * DH].astype(jnp.bfloat16)
                s = lax.dot_general(qh, kc[c][h], (((1,), (1,)), ((), ())),
                                    preferred_element_type=jnp.float32)
                w = jnp.exp(s - jnp.max(s, axis=1, keepdims=True))
                w = (w / jnp.sum(w, axis=1, keepdims=True)).astype(jnp.bfloat16)
                ctx = jnp.dot(w, vc[c][h], preferred_element_type=jnp.float32
                              ).astype(jnp.bfloat16)
                hc = slice(h * DH, (h + 1) * DH)
                ctx_ref[base + 64 * c:base + 64 * c + 64, hc] = ctx[:64, :]
                ctx_ref[base + 64 * c + 256:base + 64 * c + 320, hc] = ctx[64:, :]
        acc_ref[base:base + HALF, :] = jnp.dot(
            ctx_ref[base:base + HALF, :], wo_b,
            preferred_element_type=jnp.float32).astype(jnp.bfloat16)
        for p in ((0,) if h_half == 0 else (1, 2)):
            rs[p * 3] = rs_rdma(p, 0)
            rs[p * 3].start()

    for r in range(3):
        for p in range(3):
            rs[p * 3 + r].wait()
            rs_add(p, r)
            if r < 2:
                rs[p * 3 + r + 1] = rs_rdma(p, r + 1)
                rs[p * 3 + r + 1].start()

    ag = [None] * 9
    for p in range(3):
        sl = pl.ds(lo[p], SEG[p][2])
        out_ref[sl, :] = acc_ref[sl, :]
        ag[p * 3 + 2] = ag_rdma(p, 2)
        ag[p * 3 + 2].start()
    for r in (2, 1, 0):
        for p in range(3):
            ag[p * 3 + r].wait()
            if r > 0:
                ag[p * 3 + r - 1] = ag_rdma(p, r - 1)
                ag[p * 3 + r - 1].start()


def kernel(x, Wq, K_ext, V_ext, Wo):
    out = pl.pallas_call(
        _body,
        out_shape=jax.ShapeDtypeStruct((SQ, D_MODEL), jnp.bfloat16),
        in_specs=[
            pl.BlockSpec(memory_space=pltpu.VMEM),
            pl.BlockSpec(memory_space=pltpu.VMEM),
            pl.BlockSpec(memory_space=pl.ANY),
            pl.BlockSpec(memory_space=pl.ANY),
            pl.BlockSpec(memory_space=pltpu.VMEM),
        ],
        out_specs=pl.BlockSpec(memory_space=pltpu.VMEM),
        scratch_shapes=[
            pltpu.VMEM((SQ, D_MODEL), jnp.bfloat16),
            pltpu.VMEM((SQ, HQ * DH), jnp.bfloat16),
            pltpu.VMEM((HQ, SQ, DH), jnp.float32),
            pltpu.VMEM((HQ, SQ, DH), jnp.float32),
            pltpu.VMEM((SEG[0][0], D_MODEL), jnp.bfloat16),
            pltpu.VMEM((SEG[0][1], D_MODEL), jnp.bfloat16),
            pltpu.VMEM((SEG[0][2], D_MODEL), jnp.bfloat16),
            pltpu.VMEM((SEG[1][0], D_MODEL), jnp.bfloat16),
            pltpu.VMEM((SEG[1][1], D_MODEL), jnp.bfloat16),
            pltpu.VMEM((SEG[1][2], D_MODEL), jnp.bfloat16),
            pltpu.VMEM((SEG[2][0], D_MODEL), jnp.bfloat16),
            pltpu.VMEM((SEG[2][1], D_MODEL), jnp.bfloat16),
            pltpu.VMEM((SEG[2][2], D_MODEL), jnp.bfloat16),
            pltpu.SemaphoreType.DMA((HQ,)),
            pltpu.SemaphoreType.DMA((HQ,)),
            pltpu.SemaphoreType.DMA((9,)),
            pltpu.SemaphoreType.DMA((9,)),
            pltpu.SemaphoreType.DMA((9,)),
            pltpu.SemaphoreType.DMA((9,)),
        ],
        compiler_params=pltpu.CompilerParams(collective_id=0),
    )(x[0], Wq, K_ext, V_ext, Wo)
    return out[None]
